# baseline (speedup 1.0000x reference)
import functools

import jax
import jax.numpy as jnp
from jax import lax
from jax.experimental import pallas as pl
from jax.experimental.pallas import tpu as pltpu
from jax.experimental.pallas import tpu_sc as plsc

E = 8
TOPK = 2
D = 1024
DFF = 4096
T = 2048
TM = 256
TMG = 256
NT = T // TM
NI = T * TOPK
TOTAL_TILES = 24
TOTAL_ROWS = TOTAL_TILES * TMG
BF = 512
NJ = DFF // BF
NGU = (2 * DFF) // BF

_f32 = jnp.float32
_i32 = jnp.int32


def _prologue_body(hid_ref, res_ref, lnw_ref, lnb_ref, rw_ref,
                   h_ref, normed_ref, w_ref, ids_ref):
    h = hid_ref[...] + res_ref[...]
    h_ref[...] = h
    mu = jnp.mean(h, axis=-1, keepdims=True)
    var = jnp.mean((h - mu) ** 2, axis=-1, keepdims=True)
    normed = (h - mu) * lax.rsqrt(var + 1e-5) * lnw_ref[...][None, :] + lnb_ref[...][None, :]
    normed_ref[...] = normed
    logits = jax.lax.dot_general(normed, rw_ref[...],
                                 (((1,), (1,)), ((), ())),
                                 preferred_element_type=_f32)
    m = jnp.max(logits, axis=-1, keepdims=True)
    ex = jnp.exp(logits - m)
    p = ex / jnp.sum(ex, axis=-1, keepdims=True)
    i0 = jnp.argmax(p, axis=-1)
    v0 = jnp.max(p, axis=-1)
    cols = lax.broadcasted_iota(_i32, (TM, E), 1)
    pm = jnp.where(cols == i0[:, None], -jnp.inf, p)
    i1 = jnp.argmax(pm, axis=-1)
    v1 = jnp.max(pm, axis=-1)
    s = v0 + v1
    w_ref[...] = jnp.concatenate([(v0 / s)[:, None], (v1 / s)[:, None]], axis=-1)
    ids_ref[...] = jnp.concatenate([i0[:, None], i1[:, None]], axis=-1).astype(_i32)


_prologue = pl.pallas_call(
    _prologue_body,
    grid=(NT,),
    in_specs=[
        pl.BlockSpec((TM, D), lambda i: (i, 0)),
        pl.BlockSpec((TM, D), lambda i: (i, 0)),
        pl.BlockSpec((D,), lambda i: (0,)),
        pl.BlockSpec((D,), lambda i: (0,)),
        pl.BlockSpec((E, D), lambda i: (0, 0)),
    ],
    out_specs=[
        pl.BlockSpec((TM, D), lambda i: (i, 0)),
        pl.BlockSpec((TM, D), lambda i: (i, 0)),
        pl.BlockSpec((TM, TOPK), lambda i: (i, 0)),
        pl.BlockSpec((TM, TOPK), lambda i: (i, 0)),
    ],
    out_shape=[
        jax.ShapeDtypeStruct((T, D), _f32),
        jax.ShapeDtypeStruct((T, D), _f32),
        jax.ShapeDtypeStruct((T, TOPK), _f32),
        jax.ShapeDtypeStruct((T, TOPK), _i32),
    ],
)


def _route_body(ids_ref, pos_ref, tile_e_ref):
    ids = ids_ref[...]
    el = lax.broadcasted_iota(_i32, (T, E), 1)
    oh0 = (ids[:, 0:1] == el)
    oh1 = (ids[:, 1:2] == el)
    oh = oh0.astype(_f32) + oh1.astype(_f32)
    C = 512
    sltri = (lax.broadcasted_iota(_i32, (C, C), 1)
             < lax.broadcasted_iota(_i32, (C, C), 0)).astype(_f32)
    carry = jnp.zeros((1, E), _f32)
    chunks = []
    for c in range(T // C):
        blk = oh[c * C:(c + 1) * C, :]
        ex = lax.dot_general(sltri, blk, (((1,), (0,)), ((), ())),
                             preferred_element_type=_f32) + carry
        carry = carry + jnp.sum(blk, axis=0, keepdims=True)
        chunks.append(ex)
    excl = jnp.concatenate(chunks, axis=0)
    counts = carry
    tiles_f = jnp.floor((counts + (TMG - 1)) / TMG)
    ut = (lax.broadcasted_iota(_i32, (E, E), 0)
          <= lax.broadcasted_iota(_i32, (E, E), 1)).astype(_f32)
    t8 = lax.dot_general(tiles_f, ut, (((1,), (0,)), ((), ())),
                         preferred_element_type=_f32)
    pad_off = (t8 - tiles_f) * TMG
    rank0 = jnp.sum(jnp.where(oh0, excl, 0.0), axis=-1, keepdims=True)
    rank1 = jnp.sum(jnp.where(oh1, excl + oh0.astype(_f32), 0.0),
                    axis=-1, keepdims=True)
    po0 = jnp.sum(jnp.where(oh0, pad_off, 0.0), axis=-1, keepdims=True)
    po1 = jnp.sum(jnp.where(oh1, pad_off, 0.0), axis=-1, keepdims=True)
    pos = jnp.concatenate([rank0 + po0, rank1 + po1], axis=-1)
    pos_ref[...] = pos.astype(_i32)
    eye = (lax.broadcasted_iota(_i32, (E, E), 0)
           == lax.broadcasted_iota(_i32, (E, E), 1)).astype(_f32)
    t8_col = lax.dot_general(eye, t8, (((1,), (1,)), ((), ())),
                             preferred_element_type=_f32)
    tiles_col = lax.dot_general(eye, tiles_f, (((1,), (1,)), ((), ())),
                                preferred_element_type=_f32)
    present = (tiles_col > 0.0).astype(_f32)
    jj = lax.broadcasted_iota(_i32, (E, 128), 1).astype(_f32)
    te = jnp.sum((jj >= t8_col).astype(_i32), axis=0)
    te = jnp.minimum(te, E - 1)
    dcnt = jnp.sum(((jj - 32.0) >= t8_col).astype(_f32) * present,
                   axis=0).astype(_i32)
    sltE = (lax.broadcasted_iota(_i32, (E, E), 1)
            < lax.broadcasted_iota(_i32, (E, E), 0)).astype(_f32)
    rank_col = lax.dot_general(sltE, present, (((1,), (0,)), ((), ())),
                               preferred_element_type=_f32)
    e_col = lax.broadcasted_iota(_i32, (E, 1), 0).astype(_f32)
    pe = jnp.sum(((jj - 96.0) == rank_col).astype(_f32) * present * e_col,
                 axis=0).astype(_i32)
    num_tiles = jnp.sum(tiles_f).astype(_i32)
    kpresent = jnp.sum(present).astype(_i32)
    lane = lax.broadcasted_iota(_i32, (128,), 0)
    val = te
    val = jnp.where(jnp.logical_and(lane >= 32, lane < 64), dcnt, val)
    val = jnp.where(lane == 64, num_tiles, val)
    val = jnp.where(lane == 65, kpresent, val)
    val = jnp.where(lane >= 96, pe, val)
    tile_e_ref[...] = val


_route = pl.pallas_call(
    _route_body,
    grid=(1,),
    in_specs=[pl.BlockSpec((T, TOPK), lambda i: (0, 0))],
    out_specs=[
        pl.BlockSpec((T, TOPK), lambda i: (0, 0)),
        pl.BlockSpec((128,), lambda i: (0,)),
    ],
    out_shape=[
        jax.ShapeDtypeStruct((T, TOPK), _i32),
        jax.ShapeDtypeStruct((128,), _i32),
    ],
)


_CHUNK = 64
_NW = 32
_ROWS_PER_W = NI // _NW
_NCH = _ROWS_PER_W // _CHUNK

@functools.cache
def _build_scatter_x():
    mesh = plsc.VectorSubcoreMesh(core_axis_name="c", subcore_axis_name="s")

    @functools.partial(
        pl.kernel,
        out_type=jax.ShapeDtypeStruct((TOTAL_ROWS, D), _f32),
        mesh=mesh,
        scratch_types=[
            pltpu.VMEM((_CHUNK,), _i32),
            pltpu.VMEM((_CHUNK,), _i32),
            pltpu.VMEM((_CHUNK, D), _f32),
            pltpu.SemaphoreType.DMA,
        ],
    )
    def _scatter_x(normed_hbm, tok_hbm, pos_hbm, out_hbm, idx_v, pos_v, rows_v, sem):
        wid = lax.axis_index("s") * 2 + lax.axis_index("c")
        for c in range(_NCH):
            base = wid * _ROWS_PER_W + c * _CHUNK
            pltpu.sync_copy(tok_hbm.at[pl.ds(base, _CHUNK)], idx_v)
            pltpu.sync_copy(pos_hbm.at[pl.ds(base, _CHUNK)], pos_v)
            pltpu.async_copy(normed_hbm.at[idx_v], rows_v, sem).wait()
            pltpu.async_copy(rows_v, out_hbm.at[pos_v], sem).wait()

    return _scatter_x


def _gmm_body(te_ref, x_ref, gu_hbm, d_hbm, o_ref,
              acc_ref, gbuf, ubuf, dbuf, g16, u16, d16, sems):
    j = pl.program_id(0)
    m = pl.program_id(1)
    nt = te_ref[64]
    kp = te_ref[65]
    dm = te_ref[32 + m]
    dprev = te_ref[32 + jnp.maximum(m - 1, 0)]
    is_new = jnp.logical_or(m == 0, dm != dprev)
    valid = m < nt
    p = jnp.mod(j * kp + dm, 2)
    wrap = (dm + 1) >= kp
    jn = jnp.where(wrap, j + 1, j)
    en = te_ref[96 + jnp.where(wrap, 0, dm + 1)]

    def _start(e_, j_, slot):
        pltpu.make_async_copy(gu_hbm.at[e_, pl.ds(j_ * BF, BF), :],
                              gbuf.at[slot], sems.at[0, slot]).start()
        pltpu.make_async_copy(gu_hbm.at[e_, pl.ds(DFF + j_ * BF, BF), :],
                              ubuf.at[slot], sems.at[1, slot]).start()
        pltpu.make_async_copy(d_hbm.at[e_, :, pl.ds(j_ * BF, BF)],
                              dbuf.at[slot], sems.at[2, slot]).start()

    def _wait(slot):
        pltpu.make_async_copy(gu_hbm.at[0, pl.ds(0, BF), :],
                              gbuf.at[slot], sems.at[0, slot]).wait()
        pltpu.make_async_copy(gu_hbm.at[0, pl.ds(0, BF), :],
                              ubuf.at[slot], sems.at[1, slot]).wait()
        pltpu.make_async_copy(d_hbm.at[0, :, pl.ds(0, BF)],
                              dbuf.at[slot], sems.at[2, slot]).wait()

    @pl.when(jnp.logical_and(valid, is_new))
    def _():
        @pl.when(jnp.logical_and(j == 0, m == 0))
        def _():
            _start(te_ref[0], 0, p)
        _wait(p)
        g16[pl.ds(p, 1)] = gbuf[pl.ds(p, 1)].astype(jnp.bfloat16)
        u16[pl.ds(p, 1)] = ubuf[pl.ds(p, 1)].astype(jnp.bfloat16)
        d16[pl.ds(p, 1)] = dbuf[pl.ds(p, 1)].astype(jnp.bfloat16)

        @pl.when(jn < NJ)
        def _():
            _start(en, jn, 1 - p)

    @pl.when(valid)
    def _():
        row = pl.multiple_of(m * TMG, TMG)
        x = x_ref[pl.ds(row, TMG), :]
        g = lax.dot_general(x, g16[p], (((1,), (1,)), ((), ())),
                            preferred_element_type=_f32)
        u = lax.dot_general(x, u16[p], (((1,), (1,)), ((), ())),
                            preferred_element_type=_f32)
        hb = (g * jax.nn.sigmoid(g) * u).astype(jnp.bfloat16)
        part = lax.dot_general(hb, d16[p], (((1,), (1,)), ((), ())),
                               preferred_element_type=_f32)

        @pl.when(j == 0)
        def _():
            acc_ref[pl.ds(row, TMG), :] = part

        @pl.when(jnp.logical_and(j > 0, j < NJ - 1))
        def _():
            acc_ref[pl.ds(row, TMG), :] += part

        @pl.when(j == NJ - 1)
        def _():
            o_ref[...] = acc_ref[pl.ds(row, TMG), :] + part


_gmm = pl.pallas_call(
    _gmm_body,
    grid_spec=pltpu.PrefetchScalarGridSpec(
        num_scalar_prefetch=1,
        grid=(NJ, TOTAL_TILES),
        in_specs=[
            pl.BlockSpec((TOTAL_ROWS, D), lambda j, m, te: (0, 0)),
            pl.BlockSpec(memory_space=pltpu.MemorySpace.HBM),
            pl.BlockSpec(memory_space=pltpu.MemorySpace.HBM),
        ],
        out_specs=pl.BlockSpec(
            (TMG, D), lambda j, m, te: (jnp.where(j == NJ - 1, m, 0), 0)),
        scratch_shapes=[
            pltpu.VMEM((TOTAL_ROWS, D), _f32),
            pltpu.VMEM((2, BF, D), _f32),
            pltpu.VMEM((2, BF, D), _f32),
            pltpu.VMEM((2, D, BF), _f32),
            pltpu.VMEM((2, BF, D), jnp.bfloat16),
            pltpu.VMEM((2, BF, D), jnp.bfloat16),
            pltpu.VMEM((2, D, BF), jnp.bfloat16),
            pltpu.SemaphoreType.DMA((3, 2)),
        ],
    ),
    out_shape=jax.ShapeDtypeStruct((TOTAL_ROWS, D), _f32),
    compiler_params=pltpu.CompilerParams(
        dimension_semantics=("arbitrary", "arbitrary"),
        vmem_limit_bytes=100 * 1024 * 1024,
    ),
)


@functools.cache
def _build_gather_y():
    mesh = plsc.VectorSubcoreMesh(core_axis_name="c", subcore_axis_name="s")

    @functools.partial(
        pl.kernel,
        out_type=jax.ShapeDtypeStruct((NI, D), _f32),
        mesh=mesh,
        scratch_types=[
            pltpu.VMEM((_CHUNK,), _i32),
            pltpu.VMEM((_CHUNK, D), _f32),
            pltpu.SemaphoreType.DMA,
        ],
    )
    def _gather_y(y_hbm, pos_hbm, out_hbm, pos_v, rows_v, sem):
        wid = lax.axis_index("s") * 2 + lax.axis_index("c")
        for c in range(_NCH):
            base = wid * _ROWS_PER_W + c * _CHUNK
            pltpu.sync_copy(pos_hbm.at[pl.ds(base, _CHUNK)], pos_v)
            pltpu.async_copy(y_hbm.at[pos_v], rows_v, sem).wait()
            pltpu.sync_copy(rows_v, out_hbm.at[pl.ds(base, _CHUNK)])

    return _gather_y


def _combine_body(h_ref, y_ref, w_ref, o_ref):
    w = w_ref[...]
    o_ref[...] = (h_ref[...]
                  + w[:, 0:1] * y_ref[:, 0, :]
                  + w[:, 1:2] * y_ref[:, 1, :])


_combine = pl.pallas_call(
    _combine_body,
    grid=(NT,),
    in_specs=[
        pl.BlockSpec((TM, D), lambda i: (i, 0)),
        pl.BlockSpec((TM, TOPK, D), lambda i: (i, 0, 0)),
        pl.BlockSpec((TM, TOPK), lambda i: (i, 0)),
    ],
    out_specs=pl.BlockSpec((TM, D), lambda i: (i, 0)),
    out_shape=jax.ShapeDtypeStruct((T, D), _f32),
)


def kernel(hidden_states, residual, ln_weight, ln_bias, router_weight,
           gate_up_proj, down_proj):
    b, s, d = hidden_states.shape
    hs = hidden_states.reshape(T, D)
    rs = residual.reshape(T, D)
    h, normed, w2, ids = _prologue(hs, rs, ln_weight, ln_bias, router_weight)
    pos2, tile_e = _route(ids)
    pos = pos2.reshape(NI)
    tok_ids = (jnp.arange(NI, dtype=_i32) // TOPK)
    x_sorted = _build_scatter_x()(normed, tok_ids, pos)
    y = _gmm(tile_e, x_sorted.astype(jnp.bfloat16), gate_up_proj,
             down_proj)
    y_tok = _build_gather_y()(y, pos)
    out = _combine(h, y_tok.reshape(T, TOPK, D), w2)
    return out.reshape(b, s, d)


# speedup vs baseline: 1.0905x; 1.0905x over previous
import functools

import jax
import jax.numpy as jnp
from jax import lax
from jax.experimental import pallas as pl
from jax.experimental.pallas import tpu as pltpu
from jax.experimental.pallas import tpu_sc as plsc

E = 8
TOPK = 2
D = 1024
DFF = 4096
T = 2048
TM = 256
TMG = 256
NT = T // TM
NI = T * TOPK
TOTAL_TILES = 24
TOTAL_ROWS = TOTAL_TILES * TMG
BF = 512
NJ = DFF // BF
NGU = (2 * DFF) // BF

_f32 = jnp.float32
_i32 = jnp.int32


def _prologue_body(hid_ref, res_ref, lnw_ref, lnb_ref, rw_ref,
                   h_ref, normed_ref, w_ref, ids_ref):
    h = hid_ref[...] + res_ref[...]
    h_ref[...] = h
    mu = jnp.mean(h, axis=-1, keepdims=True)
    var = jnp.mean((h - mu) ** 2, axis=-1, keepdims=True)
    normed = (h - mu) * lax.rsqrt(var + 1e-5) * lnw_ref[...][None, :] + lnb_ref[...][None, :]
    normed_ref[...] = normed
    logits = jax.lax.dot_general(normed, rw_ref[...],
                                 (((1,), (1,)), ((), ())),
                                 preferred_element_type=_f32)
    m = jnp.max(logits, axis=-1, keepdims=True)
    ex = jnp.exp(logits - m)
    p = ex / jnp.sum(ex, axis=-1, keepdims=True)
    i0 = jnp.argmax(p, axis=-1)
    v0 = jnp.max(p, axis=-1)
    cols = lax.broadcasted_iota(_i32, (TM, E), 1)
    pm = jnp.where(cols == i0[:, None], -jnp.inf, p)
    i1 = jnp.argmax(pm, axis=-1)
    v1 = jnp.max(pm, axis=-1)
    s = v0 + v1
    w_ref[...] = jnp.concatenate([(v0 / s)[:, None], (v1 / s)[:, None]], axis=-1)
    ids_ref[...] = jnp.concatenate([i0[:, None], i1[:, None]], axis=-1).astype(_i32)


_prologue = pl.pallas_call(
    _prologue_body,
    grid=(NT,),
    in_specs=[
        pl.BlockSpec((TM, D), lambda i: (i, 0)),
        pl.BlockSpec((TM, D), lambda i: (i, 0)),
        pl.BlockSpec((D,), lambda i: (0,)),
        pl.BlockSpec((D,), lambda i: (0,)),
        pl.BlockSpec((E, D), lambda i: (0, 0)),
    ],
    out_specs=[
        pl.BlockSpec((TM, D), lambda i: (i, 0)),
        pl.BlockSpec((TM, D), lambda i: (i, 0)),
        pl.BlockSpec((TM, TOPK), lambda i: (i, 0)),
        pl.BlockSpec((TM, TOPK), lambda i: (i, 0)),
    ],
    out_shape=[
        jax.ShapeDtypeStruct((T, D), _f32),
        jax.ShapeDtypeStruct((T, D), _f32),
        jax.ShapeDtypeStruct((T, TOPK), _f32),
        jax.ShapeDtypeStruct((T, TOPK), _i32),
    ],
)


def _route_body(ids_ref, pos_ref, tile_e_ref):
    ids = ids_ref[...]
    el = lax.broadcasted_iota(_i32, (T, E), 1)
    oh0 = (ids[:, 0:1] == el)
    oh1 = (ids[:, 1:2] == el)
    oh = oh0.astype(_f32) + oh1.astype(_f32)
    C = 512
    sltri = (lax.broadcasted_iota(_i32, (C, C), 1)
             < lax.broadcasted_iota(_i32, (C, C), 0)).astype(_f32)
    carry = jnp.zeros((1, E), _f32)
    chunks = []
    for c in range(T // C):
        blk = oh[c * C:(c + 1) * C, :]
        ex = lax.dot_general(sltri, blk, (((1,), (0,)), ((), ())),
                             preferred_element_type=_f32) + carry
        carry = carry + jnp.sum(blk, axis=0, keepdims=True)
        chunks.append(ex)
    excl = jnp.concatenate(chunks, axis=0)
    counts = carry
    tiles_f = jnp.floor((counts + (TMG - 1)) / TMG)
    ut = (lax.broadcasted_iota(_i32, (E, E), 0)
          <= lax.broadcasted_iota(_i32, (E, E), 1)).astype(_f32)
    t8 = lax.dot_general(tiles_f, ut, (((1,), (0,)), ((), ())),
                         preferred_element_type=_f32)
    pad_off = (t8 - tiles_f) * TMG
    rank0 = jnp.sum(jnp.where(oh0, excl, 0.0), axis=-1, keepdims=True)
    rank1 = jnp.sum(jnp.where(oh1, excl + oh0.astype(_f32), 0.0),
                    axis=-1, keepdims=True)
    po0 = jnp.sum(jnp.where(oh0, pad_off, 0.0), axis=-1, keepdims=True)
    po1 = jnp.sum(jnp.where(oh1, pad_off, 0.0), axis=-1, keepdims=True)
    pos = jnp.concatenate([rank0 + po0, rank1 + po1], axis=-1)
    pos_ref[...] = pos.astype(_i32)
    eye = (lax.broadcasted_iota(_i32, (E, E), 0)
           == lax.broadcasted_iota(_i32, (E, E), 1)).astype(_f32)
    t8_col = lax.dot_general(eye, t8, (((1,), (1,)), ((), ())),
                             preferred_element_type=_f32)
    tiles_col = lax.dot_general(eye, tiles_f, (((1,), (1,)), ((), ())),
                                preferred_element_type=_f32)
    present = (tiles_col > 0.0).astype(_f32)
    jj = lax.broadcasted_iota(_i32, (E, 128), 1).astype(_f32)
    te = jnp.sum((jj >= t8_col).astype(_i32), axis=0)
    te = jnp.minimum(te, E - 1)
    dcnt = jnp.sum(((jj - 32.0) >= t8_col).astype(_f32) * present,
                   axis=0).astype(_i32)
    sltE = (lax.broadcasted_iota(_i32, (E, E), 1)
            < lax.broadcasted_iota(_i32, (E, E), 0)).astype(_f32)
    rank_col = lax.dot_general(sltE, present, (((1,), (0,)), ((), ())),
                               preferred_element_type=_f32)
    e_col = lax.broadcasted_iota(_i32, (E, 1), 0).astype(_f32)
    pe = jnp.sum(((jj - 96.0) == rank_col).astype(_f32) * present * e_col,
                 axis=0).astype(_i32)
    num_tiles = jnp.sum(tiles_f).astype(_i32)
    kpresent = jnp.sum(present).astype(_i32)
    lane = lax.broadcasted_iota(_i32, (128,), 0)
    val = te
    val = jnp.where(jnp.logical_and(lane >= 32, lane < 64), dcnt, val)
    val = jnp.where(lane == 64, num_tiles, val)
    val = jnp.where(lane == 65, kpresent, val)
    val = jnp.where(lane >= 96, pe, val)
    tile_e_ref[...] = val


_route = pl.pallas_call(
    _route_body,
    grid=(1,),
    in_specs=[pl.BlockSpec((T, TOPK), lambda i: (0, 0))],
    out_specs=[
        pl.BlockSpec((T, TOPK), lambda i: (0, 0)),
        pl.BlockSpec((128,), lambda i: (0,)),
    ],
    out_shape=[
        jax.ShapeDtypeStruct((T, TOPK), _i32),
        jax.ShapeDtypeStruct((128,), _i32),
    ],
)


_CHUNK = 64
_NW = 32
_ROWS_PER_W = NI // _NW
_NCH = _ROWS_PER_W // _CHUNK

@functools.cache
def _build_scatter_x():
    mesh = plsc.VectorSubcoreMesh(core_axis_name="c", subcore_axis_name="s")

    @functools.partial(
        pl.kernel,
        out_type=jax.ShapeDtypeStruct((TOTAL_ROWS, D), _f32),
        mesh=mesh,
        scratch_types=[
            pltpu.VMEM((_CHUNK,), _i32),
            pltpu.VMEM((_CHUNK,), _i32),
            pltpu.VMEM((_CHUNK, D), _f32),
            pltpu.SemaphoreType.DMA,
        ],
    )
    def _scatter_x(normed_hbm, tok_hbm, pos_hbm, out_hbm, idx_v, pos_v, rows_v, sem):
        wid = lax.axis_index("s") * 2 + lax.axis_index("c")
        for c in range(_NCH):
            base = wid * _ROWS_PER_W + c * _CHUNK
            pltpu.sync_copy(tok_hbm.at[pl.ds(base, _CHUNK)], idx_v)
            pltpu.sync_copy(pos_hbm.at[pl.ds(base, _CHUNK)], pos_v)
            pltpu.async_copy(normed_hbm.at[idx_v], rows_v, sem).wait()
            pltpu.async_copy(rows_v, out_hbm.at[pos_v], sem).wait()

    return _scatter_x


def _gmm_body(te_ref, x_ref, gu_hbm, d_hbm, o_ref,
              acc_ref, gbuf, ubuf, dbuf, sems):
    j = pl.program_id(0)
    m = pl.program_id(1)
    nt = te_ref[64]
    kp = te_ref[65]
    dm = te_ref[32 + m]
    dprev = te_ref[32 + jnp.maximum(m - 1, 0)]
    is_new = jnp.logical_or(m == 0, dm != dprev)
    valid = m < nt
    p = jnp.mod(j * kp + dm, 2)
    wrap = (dm + 1) >= kp
    jn = jnp.where(wrap, j + 1, j)
    en = te_ref[96 + jnp.where(wrap, 0, dm + 1)]

    def _start(e_, j_, slot):
        pltpu.make_async_copy(gu_hbm.at[e_, pl.ds(j_ * BF, BF), :],
                              gbuf.at[slot], sems.at[0, slot]).start()
        pltpu.make_async_copy(gu_hbm.at[e_, pl.ds(DFF + j_ * BF, BF), :],
                              ubuf.at[slot], sems.at[1, slot]).start()
        pltpu.make_async_copy(d_hbm.at[e_, :, pl.ds(j_ * BF, BF)],
                              dbuf.at[slot], sems.at[2, slot]).start()

    def _wait(slot):
        pltpu.make_async_copy(gu_hbm.at[0, pl.ds(0, BF), :],
                              gbuf.at[slot], sems.at[0, slot]).wait()
        pltpu.make_async_copy(gu_hbm.at[0, pl.ds(0, BF), :],
                              ubuf.at[slot], sems.at[1, slot]).wait()
        pltpu.make_async_copy(d_hbm.at[0, :, pl.ds(0, BF)],
                              dbuf.at[slot], sems.at[2, slot]).wait()

    @pl.when(jnp.logical_and(valid, is_new))
    def _():
        @pl.when(jnp.logical_and(j == 0, m == 0))
        def _():
            _start(te_ref[0], 0, p)
        _wait(p)

        @pl.when(jn < NJ)
        def _():
            _start(en, jn, 1 - p)

    @pl.when(valid)
    def _():
        row = pl.multiple_of(m * TMG, TMG)
        x = x_ref[pl.ds(row, TMG), :]
        g = lax.dot_general(x, gbuf[p], (((1,), (1,)), ((), ())),
                            preferred_element_type=_f32)
        u = lax.dot_general(x, ubuf[p], (((1,), (1,)), ((), ())),
                            preferred_element_type=_f32)
        hb = g * jax.nn.sigmoid(g) * u
        part = lax.dot_general(hb, dbuf[p], (((1,), (1,)), ((), ())),
                               preferred_element_type=_f32)

        @pl.when(j == 0)
        def _():
            acc_ref[pl.ds(row, TMG), :] = part

        @pl.when(jnp.logical_and(j > 0, j < NJ - 1))
        def _():
            acc_ref[pl.ds(row, TMG), :] += part

        @pl.when(j == NJ - 1)
        def _():
            o_ref[...] = acc_ref[pl.ds(row, TMG), :] + part


_gmm = pl.pallas_call(
    _gmm_body,
    grid_spec=pltpu.PrefetchScalarGridSpec(
        num_scalar_prefetch=1,
        grid=(NJ, TOTAL_TILES),
        in_specs=[
            pl.BlockSpec((TOTAL_ROWS, D), lambda j, m, te: (0, 0)),
            pl.BlockSpec(memory_space=pltpu.MemorySpace.HBM),
            pl.BlockSpec(memory_space=pltpu.MemorySpace.HBM),
        ],
        out_specs=pl.BlockSpec(
            (TMG, D), lambda j, m, te: (jnp.where(j == NJ - 1, m, 0), 0)),
        scratch_shapes=[
            pltpu.VMEM((TOTAL_ROWS, D), _f32),
            pltpu.VMEM((2, BF, D), _f32),
            pltpu.VMEM((2, BF, D), _f32),
            pltpu.VMEM((2, D, BF), _f32),
            pltpu.SemaphoreType.DMA((3, 2)),
        ],
    ),
    out_shape=jax.ShapeDtypeStruct((TOTAL_ROWS, D), _f32),
    compiler_params=pltpu.CompilerParams(
        dimension_semantics=("arbitrary", "arbitrary"),
        vmem_limit_bytes=100 * 1024 * 1024,
    ),
)


@functools.cache
def _build_gather_y():
    mesh = plsc.VectorSubcoreMesh(core_axis_name="c", subcore_axis_name="s")

    @functools.partial(
        pl.kernel,
        out_type=jax.ShapeDtypeStruct((NI, D), _f32),
        mesh=mesh,
        scratch_types=[
            pltpu.VMEM((_CHUNK,), _i32),
            pltpu.VMEM((_CHUNK, D), _f32),
            pltpu.SemaphoreType.DMA,
        ],
    )
    def _gather_y(y_hbm, pos_hbm, out_hbm, pos_v, rows_v, sem):
        wid = lax.axis_index("s") * 2 + lax.axis_index("c")
        for c in range(_NCH):
            base = wid * _ROWS_PER_W + c * _CHUNK
            pltpu.sync_copy(pos_hbm.at[pl.ds(base, _CHUNK)], pos_v)
            pltpu.async_copy(y_hbm.at[pos_v], rows_v, sem).wait()
            pltpu.sync_copy(rows_v, out_hbm.at[pl.ds(base, _CHUNK)])

    return _gather_y


def _combine_body(h_ref, y_ref, w_ref, o_ref):
    w = w_ref[...]
    o_ref[...] = (h_ref[...]
                  + w[:, 0:1] * y_ref[:, 0, :]
                  + w[:, 1:2] * y_ref[:, 1, :])


_combine = pl.pallas_call(
    _combine_body,
    grid=(NT,),
    in_specs=[
        pl.BlockSpec((TM, D), lambda i: (i, 0)),
        pl.BlockSpec((TM, TOPK, D), lambda i: (i, 0, 0)),
        pl.BlockSpec((TM, TOPK), lambda i: (i, 0)),
    ],
    out_specs=pl.BlockSpec((TM, D), lambda i: (i, 0)),
    out_shape=jax.ShapeDtypeStruct((T, D), _f32),
)


def kernel(hidden_states, residual, ln_weight, ln_bias, router_weight,
           gate_up_proj, down_proj):
    b, s, d = hidden_states.shape
    hs = hidden_states.reshape(T, D)
    rs = residual.reshape(T, D)
    h, normed, w2, ids = _prologue(hs, rs, ln_weight, ln_bias, router_weight)
    pos2, tile_e = _route(ids)
    pos = pos2.reshape(NI)
    tok_ids = (jnp.arange(NI, dtype=_i32) // TOPK)
    x_sorted = _build_scatter_x()(normed, tok_ids, pos)
    y = _gmm(tile_e, x_sorted, gate_up_proj, down_proj)
    y_tok = _build_gather_y()(y, pos)
    out = _combine(h, y_tok.reshape(T, TOPK, D), w2)
    return out.reshape(b, s, d)
